# Initial kernel scaffold; baseline (speedup 1.0000x reference)
import jax
import jax.numpy as jnp
from jax.experimental import pallas as pl


def kernel(nodal_values, coords, elements):
    raise NotImplementedError("write your pallas kernel here")


# speedup vs baseline: 12.6573x; 12.6573x over previous
import functools

import jax
import jax.numpy as jnp
from jax import lax
from jax.experimental import pallas as pl
from jax.experimental.pallas import tpu as pltpu
from jax.experimental.pallas import tpu_sc as plsc

NC = 2
NS = 16
NW = NC * NS
LANES = 16

GROUPS_PER_CHUNK = 16
EPC = GROUPS_PER_CHUNK * LANES
ROWS_PER_CHUNK = 3 * EPC
IDX_SLICES = ROWS_PER_CHUNK // 128


def _sc_partials(table, elem_flat, n_elements, chunks_per_tile):
    mesh = plsc.VectorSubcoreMesh(core_axis_name="c", subcore_axis_name="s")

    @functools.partial(
        pl.kernel,
        mesh=mesh,
        compiler_params=pltpu.CompilerParams(
            needs_layout_passes=False, use_tc_tiling_on_sc=False),
        out_type=jax.ShapeDtypeStruct((NW * LANES,), jnp.float32),
        scratch_types=[
            pltpu.VMEM((ROWS_PER_CHUNK,), jnp.int32),
            pltpu.VMEM((ROWS_PER_CHUNK, LANES), jnp.float32),
            pltpu.VMEM((ROWS_PER_CHUNK * 17,), jnp.float32),
            pltpu.VMEM((LANES,), jnp.float32),
            pltpu.SemaphoreType.DMA,
        ],
    )
    def sc_k(table_hbm, elem_hbm, out_hbm, idx_v, rows_v, trans_v, acc_v, sem):
        wid = lax.axis_index("s") * NC + lax.axis_index("c")
        acc_v[...] = jnp.zeros((LANES,), jnp.float32)
        iot = lax.iota(jnp.int32, LANES)

        def chunk_body(c, _):
            row0 = (wid * chunks_per_tile + c) * ROWS_PER_CHUNK
            pltpu.sync_copy(elem_hbm.at[pl.ds(row0, ROWS_PER_CHUNK)], idx_v)
            copies = []
            for j in range(IDX_SLICES):
                copies.append(
                    pltpu.async_copy(
                        table_hbm.at[idx_v.at[pl.ds(j * 128, 128)]],
                        rows_v.at[pl.ds(j * 128, 128)],
                        sem,
                    )
                )
            for cp in copies:
                cp.wait()

            def repack_body(r, _):
                for k in range(4):
                    row = rows_v[r * 4 + k]
                    trans_v[pl.ds((r * 4 + k) * 17, LANES)] = row
                return _

            lax.fori_loop(0, ROWS_PER_CHUNK // 4, repack_body, None)

            el_chunk0 = (wid * chunks_per_tile + c) * EPC
            iot51 = iot * 51

            def group_body(g, _):
                idx0 = g * (3 * LANES * 17) + iot51
                va = []
                for a in range(3):
                    va.append([
                        plsc.load_gather(trans_v, [idx0 + (17 * a + d)])
                        for d in range(10)
                    ])
                v0, v1, v2 = va
                e1x = v1[8] - v0[8]
                e1y = v1[9] - v0[9]
                e2x = v2[8] - v0[8]
                e2y = v2[9] - v0[9]
                det = e1x * e2y - e2x * e1y
                P = jnp.zeros((LANES,), jnp.float32)
                Q = jnp.zeros((LANES,), jnp.float32)
                R = jnp.zeros((LANES,), jnp.float32)
                F = jnp.zeros((LANES,), jnp.float32)
                for d in range(8):
                    g1 = v1[d] - v0[d]
                    g2 = v2[d] - v0[d]
                    P = P + g1 * g1
                    Q = Q + g1 * g2
                    R = R + g2 * g2
                    s = v0[d] + v1[d] + v2[d]
                    for q in range(3):
                        u = s * (1.0 / 6.0) + va[q][d] * 0.5
                        t = u * u
                        F = F + t * t
                A = e2x * e2x + e2y * e2y
                B = e1x * e2x + e1y * e2y
                C = e1x * e1x + e1y * e1y
                energy = (0.25 * (A * P - 2.0 * B * Q + C * R) / det
                          + det * (1.0 / 24.0) * F)
                el_id = el_chunk0 + g * LANES + iot
                energy = jnp.where(el_id < n_elements, energy,
                                   jnp.zeros((LANES,), jnp.float32))
                acc_v[...] = acc_v[...] + energy
                return _

            lax.fori_loop(0, GROUPS_PER_CHUNK, group_body, None)
            return _

        lax.fori_loop(0, chunks_per_tile, chunk_body, None)
        pltpu.sync_copy(acc_v, out_hbm.at[pl.ds(wid * LANES, LANES)])

    return sc_k(table, elem_flat)


def _tc_reduce(partials):
    def body(p_ref, o_ref):
        o_ref[...] = jnp.sum(p_ref[...], keepdims=True)

    return pl.pallas_call(
        body,
        out_shape=jax.ShapeDtypeStruct((1, 1), jnp.float32),
    )(partials)


def kernel(nodal_values, coords, elements):
    n_nodes = nodal_values.shape[0]
    n_elements = elements.shape[0]

    table = jnp.concatenate(
        [nodal_values, coords,
         jnp.zeros((n_nodes, 6), jnp.float32)], axis=1)

    per_round = NW * EPC
    e_pad = ((n_elements + per_round - 1) // per_round) * per_round
    chunks_per_tile = e_pad // per_round
    elem_flat = jnp.pad(elements.reshape(-1), (0, 3 * (e_pad - n_elements)))

    partials = _sc_partials(table, elem_flat, n_elements, chunks_per_tile)
    total = _tc_reduce(partials.reshape(NW, LANES))
    return total[0, 0]
